# baseline (speedup 1.0000x reference)
import functools

import jax
import jax.numpy as jnp
from jax.experimental import pallas as pl
from jax.experimental.pallas import tpu as pltpu

_TB = 256


def _fused_body(x_ref, g1_ref, b1_ref, g2_ref, b2_ref, gw_ref, gb_ref,
                w1_ref, eb1_ref, w2_ref, eb2_ref, smw_ref, smb_ref, out_ref,
                h_ref):
    TB = x_ref.shape[0]
    xt = x_ref[...]
    s1 = jnp.concatenate([xt[:, dv:dv + 24, :] for dv in range(5)], axis=2)
    y1 = jnp.dot(s1.reshape(TB * 24, 140), g1_ref[...],
                 preferred_element_type=jnp.float32)
    y1 = jnp.maximum(y1 + b1_ref[...], 0.0).reshape(TB, 24, 384)
    cm = jnp.maximum(y1[:, :, :192], y1[:, :, 192:])
    p = jnp.max(cm.reshape(TB, 12, 2, 192), axis=2)
    s2 = jnp.concatenate([p[:, dv:dv + 10, :] for dv in range(3)], axis=2)
    y2 = jnp.dot(s2.reshape(TB * 10, 576), g2_ref[...],
                 preferred_element_type=jnp.float32)
    y2 = jnp.maximum(y2 + b2_ref[...], 0.0).reshape(TB, 10, 320)
    for i in range(10):
        h_ref[:, i * 320:(i + 1) * 320] = y2[:, i, :]
    h = h_ref[...]
    g = jnp.dot(h, gw_ref[...], preferred_element_type=jnp.float32) + gb_ref[...]
    g = g - jnp.max(g, axis=-1, keepdims=True)
    eg = jnp.exp(g)
    gate = eg / jnp.sum(eg, axis=-1, keepdims=True)
    mn = jnp.min(gate, axis=-1, keepdims=True)
    lane = jax.lax.broadcasted_iota(jnp.int32, (TB, 5), 1)
    last_min = jnp.max(jnp.where(gate == mn, lane, -1), axis=-1, keepdims=True)
    w = jnp.where(lane == last_min, 0.0, gate)
    acc = jnp.zeros((TB, 128), jnp.float32)
    for e in range(5):
        hid = jnp.tanh(jnp.dot(h, w1_ref[e],
                               preferred_element_type=jnp.float32)
                       + eb1_ref[e:e + 1, :])
        oute = jnp.tanh(jnp.dot(hid, w2_ref[e],
                                preferred_element_type=jnp.float32)
                        + eb2_ref[e:e + 1, :])
        acc = acc + w[:, e:e + 1] * oute
    logits = jnp.dot(acc, smw_ref[...],
                     preferred_element_type=jnp.float32) + smb_ref[...]
    logits = logits - jnp.max(logits, axis=-1, keepdims=True)
    el = jnp.exp(logits)
    out_ref[...] = el / jnp.sum(el, axis=-1, keepdims=True)


@jax.jit
def kernel(x, conv1_w, conv1_b, conv2_w, conv2_b, gate_w, gate_b,
           e_w1, e_b1, e_w2, e_b2, sm_w, sm_b):
    B = x.shape[0]
    x_r = x.reshape(B, 28, 28)

    k1 = jnp.arange(140)
    kdv1, kw1 = k1 // 28, k1 % 28
    n1 = jnp.arange(384)
    par, rem = n1 // 192, n1 % 192
    o1, jp = rem // 12, rem % 12
    j1 = 2 * jp + par
    dj1 = kw1[:, None] - j1[None, :]
    ok1 = (dj1 >= 0) & (dj1 < 5)
    G1 = jnp.where(ok1, conv1_w[o1[None, :], 0, kdv1[:, None],
                                jnp.clip(dj1, 0, 4)], 0.0)
    b1v = conv1_b[o1][None, :]

    k2 = jnp.arange(576)
    kdv2 = k2 // 192
    c2 = (k2 % 192) // 12
    kw2 = k2 % 12
    n2 = jnp.arange(320)
    o2, j2 = n2 // 10, n2 % 10
    dj2 = kw2[:, None] - j2[None, :]
    ok2 = (dj2 >= 0) & (dj2 < 3)
    G2 = jnp.where(ok2, conv2_w[o2[None, :], c2[:, None], kdv2[:, None],
                                jnp.clip(dj2, 0, 2)], 0.0)
    b2v = conv2_b[o2][None, :]

    dp = jnp.arange(3200)
    i_h = dp // 320
    o_h = (dp % 320) // 10
    j_h = dp % 10
    refd = o_h * 100 + i_h * 10 + j_h
    gwp = gate_w[refd, :]
    w1p = e_w1[:, refd, :]

    grid = (B // _TB,)
    full = lambda shape: pl.BlockSpec(shape, lambda i: (0,) * len(shape))
    out = pl.pallas_call(
        _fused_body,
        grid=grid,
        in_specs=[
            pl.BlockSpec((_TB, 28, 28), lambda i: (i, 0, 0)),
            full((140, 384)), full((1, 384)),
            full((576, 320)), full((1, 320)),
            full((3200, 5)), full((1, 5)),
            full((5, 3200, 128)), full((5, 128)),
            full((5, 128, 128)), full((5, 128)),
            full((128, 10)), full((1, 10)),
        ],
        out_specs=pl.BlockSpec((_TB, 10), lambda i: (i, 0)),
        out_shape=jax.ShapeDtypeStruct((B, 10), jnp.float32),
        scratch_shapes=[pltpu.VMEM((_TB, 3200), jnp.float32)],
    )(x_r, G1, b1v, G2, b2v, gwp, gate_b[None, :], w1p, e_b1, e_w2, e_b2,
      sm_w, sm_b[None, :])
    return out


# speedup vs baseline: 1.2356x; 1.2356x over previous
import jax
import jax.numpy as jnp
from jax.experimental import pallas as pl

_TB = 256


def _fused_body(x_ref, g1_ref, b1_ref, g2_ref, b2_ref, gw_ref, gb_ref,
                w1_ref, eb1_ref, w2_ref, eb2_ref, smw_ref, smb_ref, out_ref):
    TB = x_ref.shape[0]
    xt = x_ref[...].astype(jnp.float32)
    g1 = g1_ref[...]
    pparts = []
    for ig in range(3):
        y = jnp.dot(xt[:, 224 * ig:224 * ig + 384], g1,
                    preferred_element_type=jnp.float32)
        y = jnp.maximum(y + b1_ref[...], 0.0)
        y = jnp.maximum(y[:, :1536], y[:, 1536:])
        y = jnp.maximum(y[:, :768], y[:, 768:])
        pparts.append(y)
    p = jnp.concatenate(pparts, axis=1).astype(jnp.float32)
    h = jnp.dot(p, g2_ref[...], preferred_element_type=jnp.float32)
    h = jnp.maximum(h + b2_ref[...], 0.0)
    hb = h.astype(jnp.float32)
    g = jnp.dot(hb, gw_ref[...],
                preferred_element_type=jnp.float32) + gb_ref[...]
    g = g - jnp.max(g, axis=-1, keepdims=True)
    eg = jnp.exp(g)
    gate = eg / jnp.sum(eg, axis=-1, keepdims=True)
    mn = jnp.min(gate, axis=-1, keepdims=True)
    lane = jax.lax.broadcasted_iota(jnp.int32, (TB, 5), 1)
    last_min = jnp.max(jnp.where(gate == mn, lane, -1), axis=-1, keepdims=True)
    w = jnp.where(lane == last_min, 0.0, gate)
    acc = jnp.zeros((TB, 128), jnp.float32)
    for e in range(5):
        hid = jnp.tanh(jnp.dot(hb, w1_ref[e],
                               preferred_element_type=jnp.float32)
                       + eb1_ref[e:e + 1, :])
        oute = jnp.tanh(jnp.dot(hid, w2_ref[e],
                                preferred_element_type=jnp.float32)
                        + eb2_ref[e:e + 1, :])
        acc = acc + w[:, e:e + 1] * oute
    logits = jnp.dot(acc, smw_ref[...],
                     preferred_element_type=jnp.float32) + smb_ref[...]
    logits = logits - jnp.max(logits, axis=-1, keepdims=True)
    el = jnp.exp(logits)
    out_ref[...] = el / jnp.sum(el, axis=-1, keepdims=True)


@jax.jit
def kernel(x, conv1_w, conv1_b, conv2_w, conv2_b, gate_w, gate_b,
           e_w1, e_b1, e_w2, e_b2, sm_w, sm_b):
    B = x.shape[0]
    f32 = jnp.float32
    x2d = jnp.pad(x.reshape(B, 784), ((0, 0), (0, 48)))

    w1s = conv1_w[:, 0]
    dr = jnp.arange(12)
    a1 = (dr[:, None, None, None] ==
          2 * jnp.arange(4)[None, :, None, None]
          + jnp.arange(2)[None, None, :, None]
          + jnp.arange(5)[None, None, None, :]).astype(f32)
    wv = jnp.arange(28)
    b1 = (wv[:, None, None, None] ==
          2 * jnp.arange(12)[None, None, :, None]
          + jnp.arange(2)[None, :, None, None]
          + jnp.arange(5)[None, None, None, :]).astype(f32)
    G1 = jnp.einsum('ovd,Dtrv,wpjd->Dwprtoj', w1s, a1, b1)
    G1 = G1.reshape(336, 3072)
    G1 = jnp.pad(G1, ((0, 48), (0, 0))).astype(jnp.float32)
    b1v = jnp.broadcast_to(conv1_b[None, None, None, :, None],
                           (2, 2, 4, 16, 12)).reshape(1, 3072)

    a2 = (jnp.arange(12)[:, None, None] == jnp.arange(10)[None, :, None]
          + jnp.arange(3)[None, None, :]).astype(f32)
    G2 = jnp.einsum('ocvd,PIv,wJd->PcwoIJ', conv2_w, a2, a2)
    G2 = G2.reshape(2304, 3200).astype(jnp.float32)
    b2v = jnp.broadcast_to(conv2_b[:, None, None],
                           (32, 10, 10)).reshape(1, 3200)

    grid = (B // _TB,)
    full = lambda shape: pl.BlockSpec(shape, lambda i: (0,) * len(shape))
    out = pl.pallas_call(
        _fused_body,
        grid=grid,
        in_specs=[
            pl.BlockSpec((_TB, 832), lambda i: (i, 0)),
            full((384, 3072)), full((1, 3072)),
            full((2304, 3200)), full((1, 3200)),
            full((3200, 5)), full((1, 5)),
            full((5, 3200, 128)), full((5, 128)),
            full((5, 128, 128)), full((5, 128)),
            full((128, 10)), full((1, 10)),
        ],
        out_specs=pl.BlockSpec((_TB, 10), lambda i: (i, 0)),
        out_shape=jax.ShapeDtypeStruct((B, 10), jnp.float32),
    )(x2d, G1, b1v, G2, b2v, gate_w.astype(jnp.float32), gate_b[None, :],
      e_w1.astype(jnp.float32), e_b1, e_w2, e_b2, sm_w, sm_b[None, :])
    return out


# speedup vs baseline: 2.7377x; 2.2157x over previous
import jax
import jax.numpy as jnp
from jax.experimental import pallas as pl

_TB = 256


def _fused_body(x_ref, g1_ref, b1_ref, g2_ref, b2_ref, gw_ref, gb_ref,
                w1_ref, eb1_ref, w2_ref, eb2_ref, smw_ref, smb_ref, out_ref):
    TB = x_ref.shape[0]
    xt = x_ref[...].astype(jnp.float32)
    g1 = g1_ref[...]
    pparts = []
    for ig in range(3):
        y = jnp.dot(xt[:, 224 * ig:224 * ig + 384], g1,
                    preferred_element_type=jnp.float32)
        y = jnp.maximum(y + b1_ref[...], 0.0)
        y = jnp.maximum(y[:, :1536], y[:, 1536:])
        y = jnp.maximum(y[:, :768], y[:, 768:])
        pparts.append(y)
    p = jnp.concatenate(pparts, axis=1).astype(jnp.float32)
    h = jnp.dot(p, g2_ref[...], preferred_element_type=jnp.float32)
    h = jnp.maximum(h + b2_ref[...], 0.0)
    hb = h.astype(jnp.float32)
    g = jnp.dot(hb, gw_ref[...],
                preferred_element_type=jnp.float32) + gb_ref[...]
    g = g - jnp.max(g, axis=-1, keepdims=True)
    eg = jnp.exp(g)
    gate = eg / jnp.sum(eg, axis=-1, keepdims=True)
    mn = jnp.min(gate, axis=-1, keepdims=True)
    lane = jax.lax.broadcasted_iota(jnp.int32, (TB, 5), 1)
    last_min = jnp.max(jnp.where(gate == mn, lane, -1), axis=-1, keepdims=True)
    w = jnp.where(lane == last_min, 0.0, gate)
    acc = jnp.zeros((TB, 128), jnp.float32)
    for e in range(5):
        hid = jnp.tanh(jnp.dot(hb, w1_ref[e],
                               preferred_element_type=jnp.float32)
                       + eb1_ref[e:e + 1, :])
        oute = jnp.tanh(jnp.dot(hid, w2_ref[e],
                                preferred_element_type=jnp.float32)
                        + eb2_ref[e:e + 1, :])
        acc = acc + w[:, e:e + 1] * oute
    logits = jnp.dot(acc, smw_ref[...],
                     preferred_element_type=jnp.float32) + smb_ref[...]
    logits = logits - jnp.max(logits, axis=-1, keepdims=True)
    el = jnp.exp(logits)
    out_ref[...] = el / jnp.sum(el, axis=-1, keepdims=True)


@jax.jit
def kernel(x, conv1_w, conv1_b, conv2_w, conv2_b, gate_w, gate_b,
           e_w1, e_b1, e_w2, e_b2, sm_w, sm_b):
    B = x.shape[0]
    f32 = jnp.float32
    x2d = jnp.pad(x.reshape(B, 784), ((0, 0), (0, 48)))

    w1s = conv1_w[:, 0]
    wv = jnp.arange(28)
    bases = []
    for par in range(2):
        c1 = (wv[:, None, None] == 2 * jnp.arange(12)[None, :, None] + par
              + jnp.arange(5)[None, None, :]).astype(f32)
        bases.append(jnp.einsum('ovd,wjd->vwoj', w1s, c1).reshape(140, 192))
    blocks = []
    for par in range(2):
        for rbit in range(2):
            for t in range(4):
                r = 28 * (2 * t + rbit)
                blocks.append(jnp.pad(bases[par], ((r, 196 - r), (0, 0))))
    G1 = jnp.concatenate(blocks, axis=1)
    G1 = jnp.pad(G1, ((0, 48), (0, 0)))
    b1v = jnp.broadcast_to(conv1_b[None, None, None, :, None],
                           (2, 2, 4, 16, 12)).reshape(1, 3072)

    b2 = (jnp.arange(12)[:, None, None] == jnp.arange(10)[None, :, None]
          + jnp.arange(3)[None, None, :]).astype(f32)
    d2 = jnp.einsum('ocvd,wJd->ovcwJ', conv2_w, b2).reshape(32, 576, 10)
    blocks = []
    for o in range(32):
        do = d2[o]
        for i in range(10):
            blocks.append(jnp.pad(do, ((192 * i, 1728 - 192 * i), (0, 0))))
    G2 = jnp.concatenate(blocks, axis=1)
    b2v = jnp.broadcast_to(conv2_b[:, None, None],
                           (32, 10, 10)).reshape(1, 3200)

    grid = (B // _TB,)
    full = lambda shape: pl.BlockSpec(shape, lambda i: (0,) * len(shape))
    out = pl.pallas_call(
        _fused_body,
        grid=grid,
        in_specs=[
            pl.BlockSpec((_TB, 832), lambda i: (i, 0)),
            full((384, 3072)), full((1, 3072)),
            full((2304, 3200)), full((1, 3200)),
            full((3200, 5)), full((1, 5)),
            full((5, 3200, 128)), full((5, 128)),
            full((5, 128, 128)), full((5, 128)),
            full((128, 10)), full((1, 10)),
        ],
        out_specs=pl.BlockSpec((_TB, 10), lambda i: (i, 0)),
        out_shape=jax.ShapeDtypeStruct((B, 10), jnp.float32),
    )(x2d, G1, b1v, G2, b2v, gate_w.astype(jnp.float32), gate_b[None, :],
      e_w1.astype(jnp.float32), e_b1, e_w2, e_b2, sm_w, sm_b[None, :])
    return out


# speedup vs baseline: 6.6199x; 2.4180x over previous
import jax
import jax.numpy as jnp
from jax.experimental import pallas as pl
from jax.experimental.pallas import tpu as pltpu

_TB = 256


def _fused_body(x_ref, g1_ref, b1_ref, d2_ref, b2_ref, gw_ref, gb_ref,
                w1_ref, eb1_ref, w2_ref, eb2_ref, smw_ref, smb_ref, out_ref,
                g2_ref):
    TB = x_ref.shape[0]

    @pl.when(pl.program_id(0) == 0)
    def _build_g2():
        g2_ref[...] = jnp.zeros_like(g2_ref)
        for o in range(32):
            do = d2_ref[o]
            for i in range(10):
                g2_ref[192 * i:192 * i + 576,
                       o * 100 + i * 10:o * 100 + i * 10 + 10] = do
    xt = x_ref[...].astype(jnp.float32)
    g1 = g1_ref[...]
    pparts = []
    for ig in range(3):
        y = jnp.dot(xt[:, 224 * ig:224 * ig + 384], g1,
                    preferred_element_type=jnp.float32)
        y = jnp.maximum(y + b1_ref[...], 0.0)
        y = jnp.maximum(y[:, :1536], y[:, 1536:])
        y = jnp.maximum(y[:, :768], y[:, 768:])
        pparts.append(y)
    p = jnp.concatenate(pparts, axis=1).astype(jnp.float32)
    h = jnp.dot(p, g2_ref[...], preferred_element_type=jnp.float32)
    h = jnp.maximum(h + b2_ref[...], 0.0)
    hb = h.astype(jnp.float32)
    g = jnp.dot(hb, gw_ref[...],
                preferred_element_type=jnp.float32) + gb_ref[...]
    g = g - jnp.max(g, axis=-1, keepdims=True)
    eg = jnp.exp(g)
    gate = eg / jnp.sum(eg, axis=-1, keepdims=True)
    mn = jnp.min(gate, axis=-1, keepdims=True)
    lane = jax.lax.broadcasted_iota(jnp.int32, (TB, 5), 1)
    last_min = jnp.max(jnp.where(gate == mn, lane, -1), axis=-1, keepdims=True)
    w = jnp.where(lane == last_min, 0.0, gate)
    acc = jnp.zeros((TB, 128), jnp.float32)
    for e in range(5):
        hid = jnp.tanh(jnp.dot(hb, w1_ref[e],
                               preferred_element_type=jnp.float32)
                       + eb1_ref[e:e + 1, :])
        oute = jnp.tanh(jnp.dot(hid, w2_ref[e],
                                preferred_element_type=jnp.float32)
                        + eb2_ref[e:e + 1, :])
        acc = acc + w[:, e:e + 1] * oute
    logits = jnp.dot(acc, smw_ref[...],
                     preferred_element_type=jnp.float32) + smb_ref[...]
    logits = logits - jnp.max(logits, axis=-1, keepdims=True)
    el = jnp.exp(logits)
    out_ref[...] = el / jnp.sum(el, axis=-1, keepdims=True)


@jax.jit
def kernel(x, conv1_w, conv1_b, conv2_w, conv2_b, gate_w, gate_b,
           e_w1, e_b1, e_w2, e_b2, sm_w, sm_b):
    B = x.shape[0]
    f32 = jnp.float32
    x2d = jnp.pad(x.reshape(B, 784), ((0, 0), (0, 48)))

    w1s = conv1_w[:, 0]
    wv = jnp.arange(28)
    bases = []
    for par in range(2):
        c1 = (wv[:, None, None] == 2 * jnp.arange(12)[None, :, None] + par
              + jnp.arange(5)[None, None, :]).astype(f32)
        bases.append(jnp.einsum('ovd,wjd->vwoj', w1s, c1).reshape(140, 192))
    blocks = []
    for par in range(2):
        for rbit in range(2):
            for t in range(4):
                r = 28 * (2 * t + rbit)
                blocks.append(jnp.pad(bases[par], ((r, 196 - r), (0, 0))))
    G1 = jnp.concatenate(blocks, axis=1)
    G1 = jnp.pad(G1, ((0, 48), (0, 0)))
    b1v = jnp.broadcast_to(conv1_b[None, None, None, :, None],
                           (2, 2, 4, 16, 12)).reshape(1, 3072)

    b2 = (jnp.arange(12)[:, None, None] == jnp.arange(10)[None, :, None]
          + jnp.arange(3)[None, None, :]).astype(f32)
    d2 = jnp.einsum('ocvd,wJd->ovcwJ', conv2_w, b2).reshape(32, 576, 10)
    b2v = jnp.broadcast_to(conv2_b[:, None, None],
                           (32, 10, 10)).reshape(1, 3200)

    grid = (B // _TB,)
    full = lambda shape: pl.BlockSpec(shape, lambda i: (0,) * len(shape))
    out = pl.pallas_call(
        _fused_body,
        grid=grid,
        in_specs=[
            pl.BlockSpec((_TB, 832), lambda i: (i, 0)),
            full((384, 3072)), full((1, 3072)),
            full((32, 576, 10)), full((1, 3200)),
            full((3200, 5)), full((1, 5)),
            full((5, 3200, 128)), full((5, 128)),
            full((5, 128, 128)), full((5, 128)),
            full((128, 10)), full((1, 10)),
        ],
        out_specs=pl.BlockSpec((_TB, 10), lambda i: (i, 0)),
        out_shape=jax.ShapeDtypeStruct((B, 10), jnp.float32),
        scratch_shapes=[pltpu.VMEM((2304, 3200), jnp.float32)],
    )(x2d, G1, b1v, d2, b2v, gate_w.astype(jnp.float32), gate_b[None, :],
      e_w1.astype(jnp.float32), e_b1, e_w2, e_b2, sm_w, sm_b[None, :])
    return out
